# baseline (speedup 1.0000x reference)
import functools
import math

import jax
import jax.numpy as jnp
from jax import lax
from jax.experimental import pallas as pl
from jax.experimental.pallas import tpu as pltpu
from jax.experimental.pallas import tpu_sc as plsc

NC = 2
NS = 16
NW = NC * NS

ROWS = 2000
T = 80


def _proj_kernel(x_ref, w_ref, b_ref, q_ref, k_ref, v_ref):
    acc = jnp.dot(x_ref[...], w_ref[...],
                  preferred_element_type=jnp.float32) + b_ref[...]
    d = q_ref.shape[1]
    q_ref[...] = acc[:, :d]
    k_ref[...] = acc[:, d:2 * d]
    v_ref[...] = acc[:, 2 * d:]


def _skip_kernel(x_ref, w_ref, b_ref, s_ref):
    s_ref[...] = jnp.dot(x_ref[...], w_ref[...],
                         preferred_element_type=jnp.float32) + b_ref[...]


def _combine_kernel(n0_ref, n1_ref, dd_ref, sk_ref, o_ref):
    den = dd_ref[...] + 1e-16
    o_ref[...] = jnp.maximum(
        (n0_ref[...] + n1_ref[...]) / den + sk_ref[...], 0.0)


def _make_sc_kernel(n, e, c):
    epw = e // NW
    nchunks = epw // T
    nd = -(-n // 16) // 40 * 40 + 40
    rpt = -(-n // NS) // T * T
    rpt_last = n - (NS - 1) * rpt
    drpt = nd // NS
    assert rpt_last > 0 and rpt_last % T == 0 and nd % NS == 0
    scale = 1.0 / math.sqrt(float(c))
    nf = c // 16
    mesh = plsc.VectorSubcoreMesh(core_axis_name="c", subcore_axis_name="s",
                                  num_cores=NC, num_subcores=NS)

    @functools.partial(
        pl.kernel,
        mesh=mesh,
        out_type=[
            jax.ShapeDtypeStruct((NC, n, c), jnp.float32),
            jax.ShapeDtypeStruct((NC, nd, 128), jnp.float32),
        ],
        scratch_types=[
            pltpu.VMEM((T,), jnp.int32),
            pltpu.VMEM((T,), jnp.int32),
            pltpu.VMEM((T,), jnp.int32),
            pltpu.VMEM((T,), jnp.int32),
            pltpu.VMEM((T,), jnp.int32),
            pltpu.VMEM((T, c), jnp.float32),
            pltpu.VMEM((T, c), jnp.float32),
            pltpu.VMEM((T, c), jnp.float32),
            pltpu.VMEM((T, 128), jnp.float32),
            pltpu.VMEM((T // 16, 16), jnp.float32),
            pltpu.VMEM_SHARED((n, c), jnp.float32),
            pltpu.VMEM_SHARED((nd, 128), jnp.float32),
            pltpu.SemaphoreType.DMA,
            pltpu.SemaphoreType.DMA,
            pltpu.SemaphoreType.DMA,
            pltpu.SemaphoreType.DMA,
            pltpu.SemaphoreType.DMA,
        ],
    )
    def sc_kernel(q_hbm, k_hbm, v_hbm, src_hbm, dst_hbm,
                  numer_hbm, denom_hbm,
                  src_a, dst_a, src_b, dst_b, dgrp_idx,
                  qbuf, kbuf, vbuf, wbuf, wsm,
                  acc_n, acc_d, sem_q, sem_k, sem_v, sem_i, sem_w):
        cid = lax.axis_index("c")
        sid = lax.axis_index("s")
        wid = sid * NC + cid
        r0 = sid * rpt
        d0 = sid * drpt
        z16 = jnp.zeros((16,), jnp.float32)
        lane = lax.iota(jnp.int32, 16)

        def memset_row(i, carry2):
            for f in range(nf):
                qbuf[i, 16 * f:16 * (f + 1)] = z16
                wbuf[i, 16 * f:16 * (f + 1)] = z16
            return carry2

        lax.fori_loop(0, T, memset_row, 0)

        def zero_slab(j, carry2):
            pltpu.sync_copy(qbuf, acc_n.at[pl.ds(r0 + j * T, T)])
            return carry2

        @pl.when(sid < NS - 1)
        def _zero_main():
            lax.fori_loop(0, rpt // T, zero_slab, 0)

        @pl.when(sid == NS - 1)
        def _zero_last():
            lax.fori_loop(0, rpt_last // T, zero_slab, 0)

        pltpu.sync_copy(wbuf.at[pl.ds(0, drpt)], acc_d.at[pl.ds(d0, drpt)])
        plsc.subcore_barrier()

        ebase = wid * epw

        pltpu.sync_copy(src_hbm.at[pl.ds(ebase, T)], src_a)
        pltpu.sync_copy(dst_hbm.at[pl.ds(ebase, T)], dst_a)
        pltpu.async_copy(q_hbm.at[dst_a], qbuf, sem_q)
        pltpu.async_copy(k_hbm.at[src_a], kbuf, sem_k)
        pltpu.async_copy(v_hbm.at[src_a], vbuf, sem_v)

        def chunk(g, carry):
            @pl.when(g + 1 < nchunks)
            def _stage_next_idx():
                off = ebase + (g + 1) * T
                pltpu.async_copy(src_hbm.at[pl.ds(off, T)], src_b, sem_i)
                pltpu.async_copy(dst_hbm.at[pl.ds(off, T)], dst_b, sem_i)

            pltpu.make_async_copy(q_hbm.at[dst_a], qbuf, sem_q).wait()
            pltpu.make_async_copy(k_hbm.at[src_a], kbuf, sem_k).wait()

            @pl.when(g > 0)
            def _wait_prev_w():
                pltpu.make_async_copy(wbuf, acc_d.at[dgrp_idx], sem_w).wait()

            def group(t, carry2):
                gb = t * 16
                dvec = dst_a[pl.ds(gb, 16)]
                dgrp_idx[pl.ds(gb, 16)] = dvec >> 4
                dslot = dvec & 15
                wgrp = z16
                for j in range(16):
                    i = gb + j
                    dot = qbuf[i, 0:16] * kbuf[i, 0:16]
                    for f in range(1, nf):
                        dot = dot + (qbuf[i, 16 * f:16 * (f + 1)] *
                                     kbuf[i, 16 * f:16 * (f + 1)])
                    for m in (1, 2, 4, 8):
                        dot = dot + dot.at[lane ^ m].get(
                            mode="promise_in_bounds")
                    wv = jnp.exp(dot * scale)
                    wgrp = jnp.where(lane == j, wv, wgrp)
                    slot = dslot.at[jnp.full((16,), j, jnp.int32)].get(
                        mode="promise_in_bounds")
                    for f in range(8):
                        m8 = ((lane + 16 * f) >> 3) == slot
                        wbuf[i, 16 * f:16 * (f + 1)] = jnp.where(m8, wv, z16)
                wsm[t, :] = wgrp
                return carry2

            lax.fori_loop(0, T // 16, group, 0)

            @pl.when(g + 1 < nchunks)
            def _fire_qk():
                off = ebase + (g + 1) * T
                pltpu.make_async_copy(
                    src_hbm.at[pl.ds(off, T)], src_b, sem_i).wait()
                pltpu.make_async_copy(
                    dst_hbm.at[pl.ds(off, T)], dst_b, sem_i).wait()
                pltpu.async_copy(q_hbm.at[dst_b], qbuf, sem_q)
                pltpu.async_copy(k_hbm.at[src_b], kbuf, sem_k)

            pltpu.make_async_copy(v_hbm.at[src_a], vbuf, sem_v).wait()

            def scale_group(t, carry2):
                gb = t * 16
                wgrp = wsm[t, :]
                for j in range(16):
                    i = gb + j
                    wv = wgrp.at[jnp.full((16,), j, jnp.int32)].get(
                        mode="promise_in_bounds")
                    for f in range(nf):
                        vbuf[i, 16 * f:16 * (f + 1)] = (
                            wv * vbuf[i, 16 * f:16 * (f + 1)])
                return carry2

            lax.fori_loop(0, T // 16, scale_group, 0)

            pltpu.async_copy(wbuf, acc_d.at[dgrp_idx], sem_w, add=True)
            pltpu.sync_copy(vbuf, acc_n.at[dst_a], add=True)

            @pl.when(g + 1 < nchunks)
            def _rotate_and_fire_v():
                for t in range(T // 16):
                    src_a[pl.ds(t * 16, 16)] = src_b[pl.ds(t * 16, 16)]
                    dst_a[pl.ds(t * 16, 16)] = dst_b[pl.ds(t * 16, 16)]
                pltpu.async_copy(v_hbm.at[src_a], vbuf, sem_v)

            return carry

        lax.fori_loop(0, nchunks, chunk, 0)
        pltpu.make_async_copy(wbuf, acc_d.at[dgrp_idx], sem_w).wait()
        plsc.subcore_barrier()

        def write_slab(j, carry2):
            rr = r0 + j * T
            pltpu.sync_copy(acc_n.at[pl.ds(rr, T)], qbuf)
            pltpu.sync_copy(qbuf, numer_hbm.at[cid, pl.ds(rr, T)])
            return carry2

        @pl.when(sid < NS - 1)
        def _write_main():
            lax.fori_loop(0, rpt // T, write_slab, 0)

        @pl.when(sid == NS - 1)
        def _write_last():
            lax.fori_loop(0, rpt_last // T, write_slab, 0)

        pltpu.sync_copy(acc_d.at[pl.ds(d0, drpt)], wbuf.at[pl.ds(0, drpt)])
        pltpu.sync_copy(wbuf.at[pl.ds(0, drpt)],
                        denom_hbm.at[cid, pl.ds(d0, drpt)])

    return sc_kernel


def kernel(x, edge_index, batch, Wq, bq, Wk, bk, Wv, bv, Wskip, bskip):
    n, d = x.shape
    c = Wq.shape[1]
    e = edge_index.shape[1]

    w_all = jnp.concatenate([Wq, Wk, Wv], axis=1)
    b_all = jnp.concatenate([bq, bk, bv])[None, :]

    nb = n // ROWS
    q, k, v = pl.pallas_call(
        _proj_kernel,
        grid=(nb,),
        in_specs=[
            pl.BlockSpec((ROWS, d), lambda i: (i, 0)),
            pl.BlockSpec((d, 3 * c), lambda i: (0, 0)),
            pl.BlockSpec((1, 3 * c), lambda i: (0, 0)),
        ],
        out_specs=[
            pl.BlockSpec((ROWS, c), lambda i: (i, 0)),
            pl.BlockSpec((ROWS, c), lambda i: (i, 0)),
            pl.BlockSpec((ROWS, c), lambda i: (i, 0)),
        ],
        out_shape=[
            jax.ShapeDtypeStruct((n, c), jnp.float32),
            jax.ShapeDtypeStruct((n, c), jnp.float32),
            jax.ShapeDtypeStruct((n, c), jnp.float32),
        ],
    )(x, w_all, b_all)

    skip = pl.pallas_call(
        _skip_kernel,
        grid=(nb,),
        in_specs=[
            pl.BlockSpec((ROWS, d), lambda i: (i, 0)),
            pl.BlockSpec((d, c), lambda i: (0, 0)),
            pl.BlockSpec((1, c), lambda i: (0, 0)),
        ],
        out_specs=pl.BlockSpec((ROWS, c), lambda i: (i, 0)),
        out_shape=jax.ShapeDtypeStruct((n, c), jnp.float32),
    )(x, Wskip, bskip[None, :])

    src = edge_index[0]
    dst = edge_index[1]

    numer, denom = _make_sc_kernel(n, e, c)(q, k, v, src, dst)

    dsum = denom[0] + denom[1]
    dflat = dsum.reshape(-1, 16, 8)[:, :, 0].reshape(-1)[:n]
    dn = dflat[:, None]

    out = pl.pallas_call(
        _combine_kernel,
        grid=(nb,),
        in_specs=[
            pl.BlockSpec((ROWS, c), lambda i: (i, 0)),
            pl.BlockSpec((ROWS, c), lambda i: (i, 0)),
            pl.BlockSpec((ROWS, 1), lambda i: (i, 0)),
            pl.BlockSpec((ROWS, c), lambda i: (i, 0)),
        ],
        out_specs=pl.BlockSpec((ROWS, c), lambda i: (i, 0)),
        out_shape=jax.ShapeDtypeStruct((n, c), jnp.float32),
    )(numer[0], numer[1], dn, skip)
    return out


# speedup vs baseline: 1.0205x; 1.0205x over previous
import functools
import math

import jax
import jax.numpy as jnp
from jax import lax
from jax.experimental import pallas as pl
from jax.experimental.pallas import tpu as pltpu
from jax.experimental.pallas import tpu_sc as plsc

NC = 2
NS = 16
NW = NC * NS

ROWS = 2000
T = 80


def _proj_kernel(x_ref, w_ref, b_ref, q_ref, k_ref, v_ref):
    acc = jnp.dot(x_ref[...], w_ref[...],
                  preferred_element_type=jnp.float32) + b_ref[...]
    d = q_ref.shape[1]
    q_ref[...] = acc[:, :d]
    k_ref[...] = acc[:, d:2 * d]
    v_ref[...] = acc[:, 2 * d:]


def _skip_kernel(x_ref, w_ref, b_ref, s_ref):
    s_ref[...] = jnp.dot(x_ref[...], w_ref[...],
                         preferred_element_type=jnp.float32) + b_ref[...]


def _combine_kernel(n0_ref, n1_ref, dd_ref, sk_ref, o_ref):
    den = dd_ref[...] + 1e-16
    o_ref[...] = jnp.maximum(
        (n0_ref[...] + n1_ref[...]) / den + sk_ref[...], 0.0)


def _make_sc_kernel(n, e, c):
    epw = e // NW
    nchunks = epw // T
    nd = -(-n // 16) // 40 * 40 + 40
    rpt = -(-n // NS) // T * T
    rpt_last = n - (NS - 1) * rpt
    drpt = nd // NS
    assert rpt_last > 0 and rpt_last % T == 0 and nd % NS == 0
    scale = 1.0 / math.sqrt(float(c))
    nf = c // 16
    mesh = plsc.VectorSubcoreMesh(core_axis_name="c", subcore_axis_name="s",
                                  num_cores=NC, num_subcores=NS)

    @functools.partial(
        pl.kernel,
        mesh=mesh,
        out_type=[
            jax.ShapeDtypeStruct((NC, n, c), jnp.float32),
            jax.ShapeDtypeStruct((NC, nd, 128), jnp.float32),
        ],
        scratch_types=[
            pltpu.VMEM((T,), jnp.int32),
            pltpu.VMEM((T,), jnp.int32),
            pltpu.VMEM((T,), jnp.int32),
            pltpu.VMEM((T,), jnp.int32),
            pltpu.VMEM((T,), jnp.int32),
            pltpu.VMEM((T, c), jnp.float32),
            pltpu.VMEM((T, c), jnp.float32),
            pltpu.VMEM((T, c), jnp.float32),
            pltpu.VMEM((T, 128), jnp.float32),
            pltpu.VMEM((T // 16, 16), jnp.float32),
            pltpu.VMEM_SHARED((n, c), jnp.float32),
            pltpu.VMEM_SHARED((nd, 128), jnp.float32),
            pltpu.SemaphoreType.DMA,
            pltpu.SemaphoreType.DMA,
            pltpu.SemaphoreType.DMA,
            pltpu.SemaphoreType.DMA,
            pltpu.SemaphoreType.DMA,
        ],
    )
    def sc_kernel(q_hbm, k_hbm, v_hbm, src_hbm, dst_hbm,
                  numer_hbm, denom_hbm,
                  src_a, dst_a, src_b, dst_b, dgrp_idx,
                  qbuf, kbuf, vbuf, wbuf, wsm,
                  acc_n, acc_d, sem_q, sem_k, sem_v, sem_i, sem_w):
        cid = lax.axis_index("c")
        sid = lax.axis_index("s")
        wid = sid * NC + cid
        r0 = sid * rpt
        d0 = sid * drpt
        z16 = jnp.zeros((16,), jnp.float32)
        lane = lax.iota(jnp.int32, 16)

        def memset_row(i, carry2):
            for f in range(nf):
                qbuf[i, 16 * f:16 * (f + 1)] = z16
                wbuf[i, 16 * f:16 * (f + 1)] = z16
            return carry2

        lax.fori_loop(0, T, memset_row, 0)

        def zero_slab(j, carry2):
            pltpu.sync_copy(qbuf, acc_n.at[pl.ds(r0 + j * T, T)])
            return carry2

        @pl.when(sid < NS - 1)
        def _zero_main():
            lax.fori_loop(0, rpt // T, zero_slab, 0)

        @pl.when(sid == NS - 1)
        def _zero_last():
            lax.fori_loop(0, rpt_last // T, zero_slab, 0)

        pltpu.sync_copy(wbuf.at[pl.ds(0, drpt)], acc_d.at[pl.ds(d0, drpt)])
        plsc.subcore_barrier()

        ebase = wid * epw

        pltpu.sync_copy(src_hbm.at[pl.ds(ebase, T)], src_a)
        pltpu.sync_copy(dst_hbm.at[pl.ds(ebase, T)], dst_a)
        pltpu.async_copy(q_hbm.at[dst_a], qbuf, sem_q)
        pltpu.async_copy(k_hbm.at[src_a], kbuf, sem_k)
        pltpu.async_copy(v_hbm.at[src_a], vbuf, sem_v)

        def chunk(g, carry):
            @pl.when(g + 1 < nchunks)
            def _stage_next_idx():
                off = ebase + (g + 1) * T
                pltpu.async_copy(src_hbm.at[pl.ds(off, T)], src_b, sem_i)
                pltpu.async_copy(dst_hbm.at[pl.ds(off, T)], dst_b, sem_i)

            pltpu.make_async_copy(q_hbm.at[dst_a], qbuf, sem_q).wait()
            pltpu.make_async_copy(k_hbm.at[src_a], kbuf, sem_k).wait()

            @pl.when(g > 0)
            def _wait_prev_w():
                pltpu.make_async_copy(wbuf, acc_d.at[dgrp_idx], sem_w).wait()

            def group(t, carry2):
                gb = t * 16
                dvec = dst_a[pl.ds(gb, 16)]
                dgrp_idx[pl.ds(gb, 16)] = dvec >> 4
                dslot = dvec & 15
                wgrp = z16
                for j in range(16):
                    i = gb + j
                    dot = qbuf[i, 0:16] * kbuf[i, 0:16]
                    for f in range(1, nf):
                        dot = dot + (qbuf[i, 16 * f:16 * (f + 1)] *
                                     kbuf[i, 16 * f:16 * (f + 1)])
                    for m in (1, 2, 4, 8):
                        dot = dot + dot.at[lane ^ m].get(
                            mode="promise_in_bounds")
                    wv = jnp.exp(dot * scale)
                    wgrp = jnp.where(lane == j, wv, wgrp)
                    slot = dslot.at[jnp.full((16,), j, jnp.int32)].get(
                        mode="promise_in_bounds")
                    for f in range(8):
                        m8 = ((lane + 16 * f) >> 3) == slot
                        wbuf[i, 16 * f:16 * (f + 1)] = jnp.where(m8, wv, z16)
                wsm[t, :] = wgrp
                return carry2

            lax.fori_loop(0, T // 16, group, 0)

            @pl.when(g + 1 < nchunks)
            def _fire_qk():
                off = ebase + (g + 1) * T
                pltpu.make_async_copy(
                    src_hbm.at[pl.ds(off, T)], src_b, sem_i).wait()
                pltpu.make_async_copy(
                    dst_hbm.at[pl.ds(off, T)], dst_b, sem_i).wait()
                pltpu.async_copy(q_hbm.at[dst_b], qbuf, sem_q)
                pltpu.async_copy(k_hbm.at[src_b], kbuf, sem_k)

            pltpu.make_async_copy(v_hbm.at[src_a], vbuf, sem_v).wait()

            def scale_group(t, carry2):
                gb = t * 16
                wgrp = wsm[t, :]
                for j in range(16):
                    i = gb + j
                    wv = wgrp.at[jnp.full((16,), j, jnp.int32)].get(
                        mode="promise_in_bounds")
                    for f in range(nf):
                        vbuf[i, 16 * f:16 * (f + 1)] = (
                            wv * vbuf[i, 16 * f:16 * (f + 1)])
                return carry2

            lax.fori_loop(0, T // 16, scale_group, 0)

            pltpu.sync_copy(vbuf, acc_n.at[dst_a], add=True)
            pltpu.async_copy(wbuf, acc_d.at[dgrp_idx], sem_w, add=True)

            @pl.when(g + 1 < nchunks)
            def _rotate_and_fire_v():
                for t in range(T // 16):
                    src_a[pl.ds(t * 16, 16)] = src_b[pl.ds(t * 16, 16)]
                    dst_a[pl.ds(t * 16, 16)] = dst_b[pl.ds(t * 16, 16)]
                pltpu.async_copy(v_hbm.at[src_a], vbuf, sem_v)

            return carry

        lax.fori_loop(0, nchunks, chunk, 0)
        pltpu.make_async_copy(wbuf, acc_d.at[dgrp_idx], sem_w).wait()
        plsc.subcore_barrier()

        def write_slab(j, carry2):
            rr = r0 + j * T
            pltpu.sync_copy(acc_n.at[pl.ds(rr, T)], qbuf)
            pltpu.sync_copy(qbuf, numer_hbm.at[cid, pl.ds(rr, T)])
            return carry2

        @pl.when(sid < NS - 1)
        def _write_main():
            lax.fori_loop(0, rpt // T, write_slab, 0)

        @pl.when(sid == NS - 1)
        def _write_last():
            lax.fori_loop(0, rpt_last // T, write_slab, 0)

        pltpu.sync_copy(acc_d.at[pl.ds(d0, drpt)], wbuf.at[pl.ds(0, drpt)])
        pltpu.sync_copy(wbuf.at[pl.ds(0, drpt)],
                        denom_hbm.at[cid, pl.ds(d0, drpt)])

    return sc_kernel


def kernel(x, edge_index, batch, Wq, bq, Wk, bk, Wv, bv, Wskip, bskip):
    n, d = x.shape
    c = Wq.shape[1]
    e = edge_index.shape[1]

    w_all = jnp.concatenate([Wq, Wk, Wv], axis=1)
    b_all = jnp.concatenate([bq, bk, bv])[None, :]

    nb = n // ROWS
    q, k, v = pl.pallas_call(
        _proj_kernel,
        grid=(nb,),
        in_specs=[
            pl.BlockSpec((ROWS, d), lambda i: (i, 0)),
            pl.BlockSpec((d, 3 * c), lambda i: (0, 0)),
            pl.BlockSpec((1, 3 * c), lambda i: (0, 0)),
        ],
        out_specs=[
            pl.BlockSpec((ROWS, c), lambda i: (i, 0)),
            pl.BlockSpec((ROWS, c), lambda i: (i, 0)),
            pl.BlockSpec((ROWS, c), lambda i: (i, 0)),
        ],
        out_shape=[
            jax.ShapeDtypeStruct((n, c), jnp.float32),
            jax.ShapeDtypeStruct((n, c), jnp.float32),
            jax.ShapeDtypeStruct((n, c), jnp.float32),
        ],
    )(x, w_all, b_all)

    skip = pl.pallas_call(
        _skip_kernel,
        grid=(nb,),
        in_specs=[
            pl.BlockSpec((ROWS, d), lambda i: (i, 0)),
            pl.BlockSpec((d, c), lambda i: (0, 0)),
            pl.BlockSpec((1, c), lambda i: (0, 0)),
        ],
        out_specs=pl.BlockSpec((ROWS, c), lambda i: (i, 0)),
        out_shape=jax.ShapeDtypeStruct((n, c), jnp.float32),
    )(x, Wskip, bskip[None, :])

    src = edge_index[0]
    dst = edge_index[1]

    numer, denom = _make_sc_kernel(n, e, c)(q, k, v, src, dst)

    dsum = denom[0] + denom[1]
    dflat = dsum.reshape(-1, 16, 8)[:, :, 0].reshape(-1)[:n]
    dn = dflat[:, None]

    out = pl.pallas_call(
        _combine_kernel,
        grid=(nb,),
        in_specs=[
            pl.BlockSpec((ROWS, c), lambda i: (i, 0)),
            pl.BlockSpec((ROWS, c), lambda i: (i, 0)),
            pl.BlockSpec((ROWS, 1), lambda i: (i, 0)),
            pl.BlockSpec((ROWS, c), lambda i: (i, 0)),
        ],
        out_specs=pl.BlockSpec((ROWS, c), lambda i: (i, 0)),
        out_shape=jax.ShapeDtypeStruct((n, c), jnp.float32),
    )(numer[0], numer[1], dn, skip)
    return out
